# baseline (speedup 1.0000x reference)
import functools

import jax
import jax.numpy as jnp
from jax import lax
from jax.experimental import pallas as pl
from jax.experimental.pallas import tpu as pltpu
from jax.experimental.pallas import tpu_sc as plsc

MAX_NB = 8
D = 128
NC, NS = 2, 16
NW = NC * NS
CHUNK = 16

_MESH = plsc.VectorSubcoreMesh(
    core_axis_name="c", subcore_axis_name="s", num_cores=NC, num_subcores=NS)


def _wid():
    return lax.axis_index("s") * NC + lax.axis_index("c")


def _prep_tables(emb, wcat, bias):
    V = emb.shape[0]

    def body(emb_ref, w_ref, b_ref, ar_ref, azh_ref, aw_ref):
        acc = jnp.dot(emb_ref[...], w_ref[...],
                      preferred_element_type=jnp.float32) + b_ref[...]
        ar_ref[...] = acc[:, :D]
        azh_ref[...] = acc[:, D:3 * D]
        aw_ref[...] = acc[:, 3 * D:]

    return pl.pallas_call(
        body,
        out_shape=(jax.ShapeDtypeStruct((V, D), jnp.float32),
                   jax.ShapeDtypeStruct((V, 2 * D), jnp.float32),
                   jax.ShapeDtypeStruct((V, D), jnp.float32)),
    )(emb, wcat, bias)


def _edge_prep(fmess_p, fnode, ar_t, azh_t, ep):
    per_w = ep // NW
    pc = 112
    n_chunks = per_w // pc

    @functools.partial(
        pl.kernel,
        out_type=(jax.ShapeDtypeStruct((ep, D), jnp.float32),
                  jax.ShapeDtypeStruct((ep, 2 * D), jnp.float32)),
        mesh=_MESH,
        scratch_types=[
            pltpu.VMEM((per_w,), jnp.int32),
            pltpu.VMEM((2, pc), jnp.int32),
            pltpu.VMEM((2, pc, D), jnp.float32),
            pltpu.VMEM((2, pc, 2 * D), jnp.float32),
            pltpu.SemaphoreType.DMA,
            pltpu.SemaphoreType.DMA,
            pltpu.SemaphoreType.DMA,
            pltpu.SemaphoreType.DMA,
            pltpu.SemaphoreType.DMA,
            pltpu.SemaphoreType.DMA,
        ],
    )
    def k(fmess_hbm, fnode_hbm, art_hbm, azht_hbm, ar_hbm, azh_hbm,
          fm_v, wid_v, ar_v, azh_v, sw0, sw1, sg0, sg1, so0, so1):
        base0 = _wid() * per_w
        sws, sgs, sos = (sw0, sw1), (sg0, sg1), (so0, so1)

        def wid_copy(c, b):
            return pltpu.make_async_copy(
                fnode_hbm.at[fm_v.at[pl.ds(c * pc, pc)]],
                wid_v.at[b], sws[b])

        def row_copies(c, b):
            return (
                pltpu.make_async_copy(art_hbm.at[wid_v.at[b]],
                                      ar_v.at[b], sgs[b]),
                pltpu.make_async_copy(azht_hbm.at[wid_v.at[b]],
                                      azh_v.at[b], sgs[b]))

        def out_copies(c, b):
            base = base0 + c * pc
            return (
                pltpu.make_async_copy(ar_v.at[b],
                                      ar_hbm.at[pl.ds(base, pc)], sos[b]),
                pltpu.make_async_copy(azh_v.at[b],
                                      azh_hbm.at[pl.ds(base, pc)], sos[b]))

        pltpu.sync_copy(fmess_hbm.at[pl.ds(base0, per_w)], fm_v)
        wid_copy(0, 0).start()
        wid_copy(0, 0).wait()
        for cp in row_copies(0, 0):
            cp.start()

        def outer(c2, _):
            for b in range(2):
                c = c2 * 2 + b
                nb = (b + 1) % 2

                @pl.when(c + 1 < n_chunks)
                def _():
                    wid_copy(c + 1, nb).start()
                    wid_copy(c + 1, nb).wait()
                    @pl.when(c + 1 >= 2)
                    def _():
                        for cp in out_copies(c - 1, nb):
                            cp.wait()
                    for cp in row_copies(c + 1, nb):
                        cp.start()

                for cp in row_copies(c, b):
                    cp.wait()
                for cp in out_copies(c, b):
                    cp.start()
            return 0

        lax.fori_loop(0, n_chunks // 2, outer, 0)
        for cp in out_copies(n_chunks - 2, 0):
            cp.wait()
        for cp in out_copies(n_chunks - 1, 1):
            cp.wait()

    return k(fmess_p, fnode, ar_t, azh_t)


def _sc_sums(mgf, ar, c_tab, ep):
    per_w = ep // NW
    n_chunks = per_w // CHUNK
    nidx = CHUNK * MAX_NB

    @functools.partial(
        pl.kernel,
        out_type=jax.ShapeDtypeStruct((ep, 2 * D), jnp.float32),
        mesh=_MESH,
        scratch_types=[
            pltpu.VMEM((per_w * MAX_NB,), jnp.int32),
            pltpu.VMEM((2, nidx, 2 * D), jnp.float32),
            pltpu.VMEM((2, CHUNK, D), jnp.float32),
            pltpu.VMEM((2, CHUNK, 2 * D), jnp.float32),
            pltpu.SemaphoreType.DMA,
            pltpu.SemaphoreType.DMA,
            pltpu.SemaphoreType.DMA,
            pltpu.SemaphoreType.DMA,
            pltpu.SemaphoreType.DMA,
            pltpu.SemaphoreType.DMA,
        ],
    )
    def k(mgf_hbm, ar_hbm, c_hbm, s_hbm,
          idx_v, rows_v, ar_v, out_v, sg0, sg1, sa0, sa1, so0, so1):
        base0 = _wid() * per_w
        sgs, sas, sos = (sg0, sg1), (sa0, sa1), (so0, so1)

        def out_copy(c, b):
            return pltpu.make_async_copy(
                out_v.at[b], s_hbm.at[pl.ds(base0 + c * CHUNK, CHUNK)],
                sos[b])

        def gather_pair(c, b):
            return (
                pltpu.make_async_copy(
                    c_hbm.at[idx_v.at[pl.ds(c * nidx, nidx)]],
                    rows_v.at[b], sgs[b]),
                pltpu.make_async_copy(
                    ar_hbm.at[pl.ds(base0 + c * CHUNK, CHUNK)],
                    ar_v.at[b], sas[b]))

        def issue(c, b):
            for cp in gather_pair(c, b):
                cp.start()

        pltpu.sync_copy(
            mgf_hbm.at[pl.ds(base0 * MAX_NB, per_w * MAX_NB)], idx_v)
        issue(0, 0)

        def outer(c2, _):
            for b in range(2):
                c = c2 * 2 + b
                nb = (b + 1) % 2

                @pl.when(c + 1 < n_chunks)
                def _():
                    issue(c + 1, nb)

                for cp in gather_pair(c, b):
                    cp.wait()

                @pl.when(c >= 2)
                def _():
                    out_copy(c - 2, b).wait()

                @plsc.parallel_loop(0, CHUNK, unroll=4)
                def _(e):
                    r0 = e * MAX_NB
                    for s in range(D // 16):
                        o = s * 16
                        nar = -ar_v[b, e, pl.ds(o, 16)]
                        acc_s = jnp.zeros((16,), jnp.float32)
                        acc_g = jnp.zeros((16,), jnp.float32)
                        for j in range(MAX_NB):
                            hrow = rows_v[b, r0 + j, pl.ds(o, 16)]
                            hu = rows_v[b, r0 + j, pl.ds(D + o, 16)]
                            u = nar - hu
                            x = jnp.exp(-jnp.abs(u))
                            q = 1.0 + x
                            r = 1.457 - 0.5 * q
                            r = r * (2.0 - q * r)
                            r = r * (2.0 - q * r)
                            wneg = x * r
                            w = jnp.where(u > 0.0, wneg, 1.0 - wneg)
                            acc_s = acc_s + hrow
                            acc_g = acc_g + hrow * w
                        out_v[b, e, pl.ds(o, 16)] = acc_s
                        out_v[b, e, pl.ds(D + o, 16)] = acc_g

                out_copy(c, b).start()
            return 0

        lax.fori_loop(0, n_chunks // 2, outer, 0)
        out_copy(n_chunks - 2, 0).wait()
        out_copy(n_chunks - 1, 1).wait()

    return k(mgf, ar, c_tab)


def _tc_dense(s_tab, azh, wz2, wh2, ur, bur, ep, pad_row, blk):
    grid = ep // blk

    def body(s_ref, azh_ref, wz2_ref, wh2_ref, ur_ref, bur_ref, out_ref):
        i = pl.program_id(0)
        sum_h = s_ref[:, :D]
        sum_g = s_ref[:, D:]
        z = jax.nn.sigmoid(azh_ref[:, :D] + jnp.dot(
            sum_h, wz2_ref[...], preferred_element_type=jnp.float32))
        pre = jnp.tanh(azh_ref[:, D:] + jnp.dot(
            sum_g, wh2_ref[...], preferred_element_type=jnp.float32))
        nh = (1.0 - z) * sum_h + z * pre
        rows = i * blk + lax.broadcasted_iota(jnp.int32, (blk, 1), 0)
        is_pad = rows == pad_row
        nh = jnp.where(is_pad, 0.0, nh)
        hu = jnp.where(is_pad, bur_ref[...],
                       jnp.dot(nh, ur_ref[...],
                               preferred_element_type=jnp.float32)
                       + bur_ref[...])
        out_ref[:, :D] = nh
        out_ref[:, D:] = hu

    wspec = pl.BlockSpec((D, D), lambda i: (0, 0))
    return pl.pallas_call(
        body,
        grid=(grid,),
        in_specs=[
            pl.BlockSpec((blk, 2 * D), lambda i: (i, 0)),
            pl.BlockSpec((blk, 2 * D), lambda i: (i, 0)),
            wspec, wspec, wspec,
            pl.BlockSpec((1, D), lambda i: (0, 0)),
        ],
        out_specs=pl.BlockSpec((blk, 2 * D), lambda i: (i, 0)),
        out_shape=jax.ShapeDtypeStruct((ep, 2 * D), jnp.float32),
    )(s_tab, azh, wz2, wh2, ur, bur)


def _sc_node(ngf, fnode_p, hpad, aw, np_):
    per_w = np_ // NW
    pc = 32
    n_chunks = per_w // pc
    nidx = pc * MAX_NB

    @functools.partial(
        pl.kernel,
        out_type=(jax.ShapeDtypeStruct((np_, D), jnp.float32),
                  jax.ShapeDtypeStruct((np_, D), jnp.float32)),
        mesh=_MESH,
        scratch_types=[
            pltpu.VMEM((per_w * MAX_NB,), jnp.int32),
            pltpu.VMEM((per_w,), jnp.int32),
            pltpu.VMEM((2, nidx, D), jnp.float32),
            pltpu.VMEM((2, pc, D), jnp.float32),
            pltpu.VMEM((2, pc, D), jnp.float32),
            pltpu.SemaphoreType.DMA,
            pltpu.SemaphoreType.DMA,
            pltpu.SemaphoreType.DMA,
            pltpu.SemaphoreType.DMA,
        ],
    )
    def k(ngf_hbm, fn_hbm, hpad_hbm, awt_hbm, s2_hbm, awn_hbm,
          idx_v, fn_v, rows_v, aw_v, out_v, sg0, sg1, so0, so1):
        base0 = _wid() * per_w
        sgs, sos = (sg0, sg1), (so0, so1)

        def gathers(c, b):
            cps = [pltpu.make_async_copy(
                awt_hbm.at[fn_v.at[pl.ds(c * pc, pc)]], aw_v.at[b], sgs[b])]
            for h in range(2):
                cps.append(pltpu.make_async_copy(
                    hpad_hbm.at[idx_v.at[pl.ds(c * nidx + h * 128, 128)]],
                    rows_v.at[b, pl.ds(h * 128, 128)], sgs[b]))
            return cps

        def out_copies(c, b):
            base = base0 + c * pc
            return (
                pltpu.make_async_copy(out_v.at[b],
                                      s2_hbm.at[pl.ds(base, pc)], sos[b]),
                pltpu.make_async_copy(aw_v.at[b],
                                      awn_hbm.at[pl.ds(base, pc)], sos[b]))

        pltpu.sync_copy(ngf_hbm.at[pl.ds(base0 * MAX_NB, per_w * MAX_NB)],
                        idx_v)
        pltpu.sync_copy(fn_hbm.at[pl.ds(base0, per_w)], fn_v)
        for cp in gathers(0, 0):
            cp.start()

        def outer(c2, _):
            for b in range(2):
                c = c2 * 2 + b
                nb = (b + 1) % 2

                @pl.when(c + 1 < n_chunks)
                def _():
                    @pl.when(c + 1 >= 2)
                    def _():
                        for cp in out_copies(c - 1, nb):
                            cp.wait()
                    for cp in gathers(c + 1, nb):
                        cp.start()

                for cp in gathers(c, b):
                    cp.wait()

                @plsc.parallel_loop(0, pc, unroll=2)
                def _(n):
                    r0 = n * MAX_NB
                    for s in range(D // 16):
                        o = s * 16
                        acc = jnp.zeros((16,), jnp.float32)
                        for j in range(MAX_NB):
                            acc = acc + rows_v[b, r0 + j, pl.ds(o, 16)]
                        out_v[b, n, pl.ds(o, 16)] = acc

                for cp in out_copies(c, b):
                    cp.start()
            return 0

        lax.fori_loop(0, n_chunks // 2, outer, 0)
        for cp in out_copies(n_chunks - 2, 0):
            cp.wait()
        for cp in out_copies(n_chunks - 1, 1):
            cp.wait()

    return k(ngf, fnode_p, hpad, aw)


def _tc_root(s2, awn, ww2, np_, blk):
    grid = np_ // blk

    def body(s_ref, aw_ref, w_ref, out_ref):
        out_ref[...] = jax.nn.relu(
            aw_ref[...] + jnp.dot(s_ref[...], w_ref[...],
                                  preferred_element_type=jnp.float32))

    return pl.pallas_call(
        body,
        grid=(grid,),
        in_specs=[
            pl.BlockSpec((blk, D), lambda i: (i, 0)),
            pl.BlockSpec((blk, D), lambda i: (i, 0)),
            pl.BlockSpec((D, D), lambda i: (0, 0)),
        ],
        out_specs=pl.BlockSpec((blk, D), lambda i: (i, 0)),
        out_shape=jax.ShapeDtypeStruct((np_, D), jnp.float32),
    )(s2, awn, ww2)


def kernel(fnode, fmess, node_graph, mess_graph, depth, embedding,
           W_z_w, W_z_b, W_r_w, U_r_w, U_r_b, W_h_w, W_h_b, W_w, W_b):
    E = fmess.shape[0]
    N = fnode.shape[0]

    def _pad_to(x, m):
        q = -x % m
        return x + q

    ep = _pad_to(E + 1, 7168)
    np_ = _pad_to(N, NW * 32 * 2)
    pad_row = ep - 1
    blk = 512
    while ep % blk or np_ % blk:
        blk //= 2

    i32 = jnp.int32
    mg = jnp.where(mess_graph == 0, pad_row, mess_graph - 1).astype(i32)
    mgf = jnp.concatenate(
        [mg.reshape(-1), jnp.full(((ep - E) * MAX_NB,), pad_row, i32)])
    ng = jnp.where(node_graph == 0, pad_row, node_graph - 1).astype(i32)
    ngf = jnp.concatenate(
        [ng.reshape(-1), jnp.full(((np_ - N) * MAX_NB,), pad_row, i32)])
    fmess_p = jnp.concatenate([fmess.astype(i32), jnp.zeros((ep - E,), i32)])
    fnode_p = jnp.concatenate([fnode.astype(i32), jnp.zeros((np_ - N,), i32)])

    wz1, wz2 = W_z_w[:D], W_z_w[D:]
    wh1, wh2 = W_h_w[:D], W_h_w[D:]
    ww1, ww2 = W_w[:D], W_w[D:]
    wcat = jnp.concatenate([W_r_w, wz1, wh1, ww1], axis=1)
    bias = jnp.concatenate(
        [jnp.zeros((D,), jnp.float32), W_z_b, W_h_b, W_b]).reshape(1, 4 * D)
    bur = U_r_b.reshape(1, D)

    ar_t, azh_t, aw_t = _prep_tables(embedding, wcat, bias)

    ar, azh = _edge_prep(fmess_p, fnode.astype(i32), ar_t, azh_t, ep)

    c0 = jnp.concatenate(
        [jnp.zeros((ep, D), jnp.float32),
         jnp.broadcast_to(U_r_b, (ep, D))], axis=1)

    def body(_, c_tab):
        s_tab = _sc_sums(mgf, ar, c_tab, ep)
        return _tc_dense(s_tab, azh, wz2, wh2, U_r_w, bur, ep, pad_row, blk)

    c_tab = lax.fori_loop(0, depth, body, c0)

    hpad = c_tab[:, :D]
    s2, awn = _sc_node(ngf, fnode_p, hpad, aw_t, np_)
    root = _tc_root(s2, awn, ww2, np_, blk)

    return c_tab[:E, :D], root[:N]


# speedup vs baseline: 2.5633x; 2.5633x over previous
import functools

import jax
import jax.numpy as jnp
from jax import lax
from jax.experimental import pallas as pl
from jax.experimental.pallas import tpu as pltpu
from jax.experimental.pallas import tpu_sc as plsc

MAX_NB = 8
D = 128
NC, NS = 2, 16
NW = NC * NS
CHUNK = 16

_MESH = plsc.VectorSubcoreMesh(
    core_axis_name="c", subcore_axis_name="s", num_cores=NC, num_subcores=NS)


def _wid():
    return lax.axis_index("s") * NC + lax.axis_index("c")


def _prep_tables(emb, wcat, bias):
    V = emb.shape[0]

    def body(emb_ref, w_ref, b_ref, ar_ref, azh_ref, aw_ref):
        acc = jnp.dot(emb_ref[...], w_ref[...],
                      preferred_element_type=jnp.float32) + b_ref[...]
        ar_ref[...] = acc[:, :D]
        azh_ref[...] = acc[:, D:3 * D]
        aw_ref[...] = acc[:, 3 * D:]

    return pl.pallas_call(
        body,
        out_shape=(jax.ShapeDtypeStruct((V, D), jnp.float32),
                   jax.ShapeDtypeStruct((V, 2 * D), jnp.float32),
                   jax.ShapeDtypeStruct((V, D), jnp.float32)),
    )(emb, wcat, bias)


def _edge_prep(fmess_p, fnode, ar_t, azh_t, ep):
    per_w = ep // NW
    pc = 112
    n_chunks = per_w // pc

    @functools.partial(
        pl.kernel,
        out_type=(jax.ShapeDtypeStruct((ep, D), jnp.float32),
                  jax.ShapeDtypeStruct((ep, 2 * D), jnp.float32)),
        mesh=_MESH,
        scratch_types=[
            pltpu.VMEM((per_w,), jnp.int32),
            pltpu.VMEM((2, pc), jnp.int32),
            pltpu.VMEM((2, pc, D), jnp.float32),
            pltpu.VMEM((2, pc, 2 * D), jnp.float32),
            pltpu.SemaphoreType.DMA,
            pltpu.SemaphoreType.DMA,
            pltpu.SemaphoreType.DMA,
            pltpu.SemaphoreType.DMA,
            pltpu.SemaphoreType.DMA,
            pltpu.SemaphoreType.DMA,
        ],
    )
    def k(fmess_hbm, fnode_hbm, art_hbm, azht_hbm, ar_hbm, azh_hbm,
          fm_v, wid_v, ar_v, azh_v, sw0, sw1, sg0, sg1, so0, so1):
        base0 = _wid() * per_w
        sws, sgs, sos = (sw0, sw1), (sg0, sg1), (so0, so1)

        def wid_copy(c, b):
            return pltpu.make_async_copy(
                fnode_hbm.at[fm_v.at[pl.ds(c * pc, pc)]],
                wid_v.at[b], sws[b])

        def row_copies(c, b):
            return (
                pltpu.make_async_copy(art_hbm.at[wid_v.at[b]],
                                      ar_v.at[b], sgs[b]),
                pltpu.make_async_copy(azht_hbm.at[wid_v.at[b]],
                                      azh_v.at[b], sgs[b]))

        def out_copies(c, b):
            base = base0 + c * pc
            return (
                pltpu.make_async_copy(ar_v.at[b],
                                      ar_hbm.at[pl.ds(base, pc)], sos[b]),
                pltpu.make_async_copy(azh_v.at[b],
                                      azh_hbm.at[pl.ds(base, pc)], sos[b]))

        pltpu.sync_copy(fmess_hbm.at[pl.ds(base0, per_w)], fm_v)
        wid_copy(0, 0).start()
        wid_copy(0, 0).wait()
        for cp in row_copies(0, 0):
            cp.start()

        def outer(c2, _):
            for b in range(2):
                c = c2 * 2 + b
                nb = (b + 1) % 2

                @pl.when(c + 1 < n_chunks)
                def _():
                    wid_copy(c + 1, nb).start()
                    wid_copy(c + 1, nb).wait()
                    @pl.when(c + 1 >= 2)
                    def _():
                        for cp in out_copies(c - 1, nb):
                            cp.wait()
                    for cp in row_copies(c + 1, nb):
                        cp.start()

                for cp in row_copies(c, b):
                    cp.wait()
                for cp in out_copies(c, b):
                    cp.start()
            return 0

        lax.fori_loop(0, n_chunks // 2, outer, 0)
        for cp in out_copies(n_chunks - 2, 0):
            cp.wait()
        for cp in out_copies(n_chunks - 1, 1):
            cp.wait()

    return k(fmess_p, fnode, ar_t, azh_t)


def _sc_sums(mgf, ar, c_tab, ep):
    per_w = ep // NW
    n_chunks = per_w // CHUNK
    nidx = CHUNK * MAX_NB

    @functools.partial(
        pl.kernel,
        out_type=jax.ShapeDtypeStruct((ep, 2 * D), jnp.float32),
        mesh=_MESH,
        scratch_types=[
            pltpu.VMEM((per_w * MAX_NB,), jnp.int32),
            pltpu.VMEM((2, nidx, 2 * D), jnp.float32),
            pltpu.VMEM((2, CHUNK, D), jnp.float32),
            pltpu.VMEM((2, CHUNK, 2 * D), jnp.float32),
            pltpu.SemaphoreType.DMA,
            pltpu.SemaphoreType.DMA,
            pltpu.SemaphoreType.DMA,
            pltpu.SemaphoreType.DMA,
            pltpu.SemaphoreType.DMA,
            pltpu.SemaphoreType.DMA,
        ],
    )
    def k(mgf_hbm, ar_hbm, c_hbm, s_hbm,
          idx_v, rows_v, ar_v, out_v, sg0, sg1, sa0, sa1, so0, so1):
        base0 = _wid() * per_w
        sgs, sas, sos = (sg0, sg1), (sa0, sa1), (so0, so1)

        def out_copy(c, b):
            return pltpu.make_async_copy(
                out_v.at[b], s_hbm.at[pl.ds(base0 + c * CHUNK, CHUNK)],
                sos[b])

        def gather_pair(c, b):
            return (
                pltpu.make_async_copy(
                    c_hbm.at[idx_v.at[pl.ds(c * nidx, nidx)]],
                    rows_v.at[b], sgs[b]),
                pltpu.make_async_copy(
                    ar_hbm.at[pl.ds(base0 + c * CHUNK, CHUNK)],
                    ar_v.at[b], sas[b]))

        def issue(c, b):
            for cp in gather_pair(c, b):
                cp.start()

        pltpu.sync_copy(
            mgf_hbm.at[pl.ds(base0 * MAX_NB, per_w * MAX_NB)], idx_v)
        issue(0, 0)

        def outer(c2, _):
            for b in range(2):
                c = c2 * 2 + b
                nb = (b + 1) % 2

                @pl.when(c + 1 < n_chunks)
                def _():
                    issue(c + 1, nb)

                for cp in gather_pair(c, b):
                    cp.wait()

                @pl.when(c >= 2)
                def _():
                    out_copy(c - 2, b).wait()

                @plsc.parallel_loop(0, CHUNK, unroll=2)
                def _(e):
                    r0 = e * MAX_NB
                    for s in range(D // 16):
                        o = s * 16
                        nar = -ar_v[b, e, pl.ds(o, 16)]
                        acc_s = jnp.zeros((16,), jnp.float32)
                        acc_g = jnp.zeros((16,), jnp.float32)
                        for j in range(MAX_NB):
                            hrow = rows_v[b, r0 + j, pl.ds(o, 16)]
                            hu = rows_v[b, r0 + j, pl.ds(D + o, 16)]
                            u = nar - hu
                            x = jnp.exp(-jnp.abs(u))
                            q = 1.0 + x
                            r = 1.457 - 0.5 * q
                            r = r * (2.0 - q * r)
                            r = r * (2.0 - q * r)
                            wneg = x * r
                            w = jnp.where(u > 0.0, wneg, 1.0 - wneg)
                            acc_s = acc_s + hrow
                            acc_g = acc_g + hrow * w
                        out_v[b, e, pl.ds(o, 16)] = acc_s
                        out_v[b, e, pl.ds(D + o, 16)] = acc_g

                out_copy(c, b).start()
            return 0

        lax.fori_loop(0, n_chunks // 2, outer, 0)
        out_copy(n_chunks - 2, 0).wait()
        out_copy(n_chunks - 1, 1).wait()

    return k(mgf, ar, c_tab)


def _tc_dense(s_tab, azh, wz2, wh2, ur, bur, ep, pad_row, blk):
    grid = ep // blk

    def body(s_ref, azh_ref, wz2_ref, wh2_ref, ur_ref, bur_ref, out_ref):
        i = pl.program_id(0)
        sum_h = s_ref[:, :D]
        sum_g = s_ref[:, D:]
        z = jax.nn.sigmoid(azh_ref[:, :D] + jnp.dot(
            sum_h, wz2_ref[...], preferred_element_type=jnp.float32))
        pre = jnp.tanh(azh_ref[:, D:] + jnp.dot(
            sum_g, wh2_ref[...], preferred_element_type=jnp.float32))
        nh = (1.0 - z) * sum_h + z * pre
        rows = i * blk + lax.broadcasted_iota(jnp.int32, (blk, 1), 0)
        is_pad = rows == pad_row
        nh = jnp.where(is_pad, 0.0, nh)
        hu = jnp.where(is_pad, bur_ref[...],
                       jnp.dot(nh, ur_ref[...],
                               preferred_element_type=jnp.float32)
                       + bur_ref[...])
        out_ref[:, :D] = nh
        out_ref[:, D:] = hu

    wspec = pl.BlockSpec((D, D), lambda i: (0, 0))
    return pl.pallas_call(
        body,
        grid=(grid,),
        in_specs=[
            pl.BlockSpec((blk, 2 * D), lambda i: (i, 0)),
            pl.BlockSpec((blk, 2 * D), lambda i: (i, 0)),
            wspec, wspec, wspec,
            pl.BlockSpec((1, D), lambda i: (0, 0)),
        ],
        out_specs=pl.BlockSpec((blk, 2 * D), lambda i: (i, 0)),
        out_shape=jax.ShapeDtypeStruct((ep, 2 * D), jnp.float32),
    )(s_tab, azh, wz2, wh2, ur, bur)


def _sc_node(ngf, fnode_p, hpad, aw, np_):
    per_w = np_ // NW
    pc = 32
    n_chunks = per_w // pc
    nidx = pc * MAX_NB

    @functools.partial(
        pl.kernel,
        out_type=(jax.ShapeDtypeStruct((np_, D), jnp.float32),
                  jax.ShapeDtypeStruct((np_, D), jnp.float32)),
        mesh=_MESH,
        scratch_types=[
            pltpu.VMEM((per_w * MAX_NB,), jnp.int32),
            pltpu.VMEM((per_w,), jnp.int32),
            pltpu.VMEM((2, nidx, D), jnp.float32),
            pltpu.VMEM((2, pc, D), jnp.float32),
            pltpu.VMEM((2, pc, D), jnp.float32),
            pltpu.SemaphoreType.DMA,
            pltpu.SemaphoreType.DMA,
            pltpu.SemaphoreType.DMA,
            pltpu.SemaphoreType.DMA,
        ],
    )
    def k(ngf_hbm, fn_hbm, hpad_hbm, awt_hbm, s2_hbm, awn_hbm,
          idx_v, fn_v, rows_v, aw_v, out_v, sg0, sg1, so0, so1):
        base0 = _wid() * per_w
        sgs, sos = (sg0, sg1), (so0, so1)

        def gathers(c, b):
            cps = [pltpu.make_async_copy(
                awt_hbm.at[fn_v.at[pl.ds(c * pc, pc)]], aw_v.at[b], sgs[b])]
            for h in range(2):
                cps.append(pltpu.make_async_copy(
                    hpad_hbm.at[idx_v.at[pl.ds(c * nidx + h * 128, 128)]],
                    rows_v.at[b, pl.ds(h * 128, 128)], sgs[b]))
            return cps

        def out_copies(c, b):
            base = base0 + c * pc
            return (
                pltpu.make_async_copy(out_v.at[b],
                                      s2_hbm.at[pl.ds(base, pc)], sos[b]),
                pltpu.make_async_copy(aw_v.at[b],
                                      awn_hbm.at[pl.ds(base, pc)], sos[b]))

        pltpu.sync_copy(ngf_hbm.at[pl.ds(base0 * MAX_NB, per_w * MAX_NB)],
                        idx_v)
        pltpu.sync_copy(fn_hbm.at[pl.ds(base0, per_w)], fn_v)
        for cp in gathers(0, 0):
            cp.start()

        def outer(c2, _):
            for b in range(2):
                c = c2 * 2 + b
                nb = (b + 1) % 2

                @pl.when(c + 1 < n_chunks)
                def _():
                    @pl.when(c + 1 >= 2)
                    def _():
                        for cp in out_copies(c - 1, nb):
                            cp.wait()
                    for cp in gathers(c + 1, nb):
                        cp.start()

                for cp in gathers(c, b):
                    cp.wait()

                @plsc.parallel_loop(0, pc, unroll=2)
                def _(n):
                    r0 = n * MAX_NB
                    for s in range(D // 16):
                        o = s * 16
                        acc = jnp.zeros((16,), jnp.float32)
                        for j in range(MAX_NB):
                            acc = acc + rows_v[b, r0 + j, pl.ds(o, 16)]
                        out_v[b, n, pl.ds(o, 16)] = acc

                for cp in out_copies(c, b):
                    cp.start()
            return 0

        lax.fori_loop(0, n_chunks // 2, outer, 0)
        for cp in out_copies(n_chunks - 2, 0):
            cp.wait()
        for cp in out_copies(n_chunks - 1, 1):
            cp.wait()

    return k(ngf, fnode_p, hpad, aw)


def _tc_root(s2, awn, ww2, np_, blk):
    grid = np_ // blk

    def body(s_ref, aw_ref, w_ref, out_ref):
        out_ref[...] = jax.nn.relu(
            aw_ref[...] + jnp.dot(s_ref[...], w_ref[...],
                                  preferred_element_type=jnp.float32))

    return pl.pallas_call(
        body,
        grid=(grid,),
        in_specs=[
            pl.BlockSpec((blk, D), lambda i: (i, 0)),
            pl.BlockSpec((blk, D), lambda i: (i, 0)),
            pl.BlockSpec((D, D), lambda i: (0, 0)),
        ],
        out_specs=pl.BlockSpec((blk, D), lambda i: (i, 0)),
        out_shape=jax.ShapeDtypeStruct((np_, D), jnp.float32),
    )(s2, awn, ww2)


def kernel(fnode, fmess, node_graph, mess_graph, depth, embedding,
           W_z_w, W_z_b, W_r_w, U_r_w, U_r_b, W_h_w, W_h_b, W_w, W_b):
    E = fmess.shape[0]
    N = fnode.shape[0]

    def _pad_to(x, m):
        q = -x % m
        return x + q

    ep = _pad_to(E + 1, 7168)
    np_ = _pad_to(N, NW * 32 * 2)
    pad_row = ep - 1
    blk = 512
    while ep % blk or np_ % blk:
        blk //= 2

    i32 = jnp.int32
    mg = jnp.where(mess_graph == 0, pad_row, mess_graph - 1).astype(i32)
    mgf = jnp.concatenate(
        [mg.reshape(-1), jnp.full(((ep - E) * MAX_NB,), pad_row, i32)])
    ng = jnp.where(node_graph == 0, pad_row, node_graph - 1).astype(i32)
    ngf = jnp.concatenate(
        [ng.reshape(-1), jnp.full(((np_ - N) * MAX_NB,), pad_row, i32)])
    fmess_p = jnp.concatenate([fmess.astype(i32), jnp.zeros((ep - E,), i32)])
    fnode_p = jnp.concatenate([fnode.astype(i32), jnp.zeros((np_ - N,), i32)])

    wz1, wz2 = W_z_w[:D], W_z_w[D:]
    wh1, wh2 = W_h_w[:D], W_h_w[D:]
    ww1, ww2 = W_w[:D], W_w[D:]
    wcat = jnp.concatenate([W_r_w, wz1, wh1, ww1], axis=1)
    bias = jnp.concatenate(
        [jnp.zeros((D,), jnp.float32), W_z_b, W_h_b, W_b]).reshape(1, 4 * D)
    bur = U_r_b.reshape(1, D)

    ar_t, azh_t, aw_t = _prep_tables(embedding, wcat, bias)

    ar, azh = _edge_prep(fmess_p, fnode.astype(i32), ar_t, azh_t, ep)

    c0 = jnp.concatenate(
        [jnp.zeros((ep, D), jnp.float32),
         jnp.broadcast_to(U_r_b, (ep, D))], axis=1)

    def body(_, c_tab):
        s_tab = _sc_sums(mgf, ar, c_tab, ep)
        return _tc_dense(s_tab, azh, wz2, wh2, U_r_w, bur, ep, pad_row, blk)

    c_tab = lax.fori_loop(0, depth, body, c0)

    hpad = c_tab[:, :D]
    s2, awn = _sc_node(ngf, fnode_p, hpad, aw_t, np_)
    root = _tc_root(s2, awn, ww2, np_, blk)

    return c_tab[:E, :D], root[:N]


# speedup vs baseline: 2.6513x; 1.0343x over previous
import functools

import jax
import jax.numpy as jnp
from jax import lax
from jax.experimental import pallas as pl
from jax.experimental.pallas import tpu as pltpu
from jax.experimental.pallas import tpu_sc as plsc

MAX_NB = 8
D = 128
NC, NS = 2, 16
NW = NC * NS
CHUNK = 16

_MESH = plsc.VectorSubcoreMesh(
    core_axis_name="c", subcore_axis_name="s", num_cores=NC, num_subcores=NS)


def _wid():
    return lax.axis_index("s") * NC + lax.axis_index("c")


def _prep_tables(emb, wcat, bias):
    V = emb.shape[0]

    def body(emb_ref, w_ref, b_ref, ar_ref, azh_ref, aw_ref):
        acc = jnp.dot(emb_ref[...], w_ref[...],
                      preferred_element_type=jnp.float32) + b_ref[...]
        ar_ref[...] = acc[:, :D]
        azh_ref[...] = acc[:, D:3 * D]
        aw_ref[...] = acc[:, 3 * D:]

    return pl.pallas_call(
        body,
        out_shape=(jax.ShapeDtypeStruct((V, D), jnp.float32),
                   jax.ShapeDtypeStruct((V, 2 * D), jnp.float32),
                   jax.ShapeDtypeStruct((V, D), jnp.float32)),
    )(emb, wcat, bias)


def _edge_prep(fmess_p, fnode, ar_t, azh_t, ep):
    per_w = ep // NW
    pc = 112
    n_chunks = per_w // pc

    @functools.partial(
        pl.kernel,
        out_type=(jax.ShapeDtypeStruct((ep, D), jnp.float32),
                  jax.ShapeDtypeStruct((ep, 2 * D), jnp.float32)),
        mesh=_MESH,
        scratch_types=[
            pltpu.VMEM((per_w,), jnp.int32),
            pltpu.VMEM((2, pc), jnp.int32),
            pltpu.VMEM((2, pc, D), jnp.float32),
            pltpu.VMEM((2, pc, 2 * D), jnp.float32),
            pltpu.SemaphoreType.DMA,
            pltpu.SemaphoreType.DMA,
            pltpu.SemaphoreType.DMA,
            pltpu.SemaphoreType.DMA,
            pltpu.SemaphoreType.DMA,
            pltpu.SemaphoreType.DMA,
        ],
    )
    def k(fmess_hbm, fnode_hbm, art_hbm, azht_hbm, ar_hbm, azh_hbm,
          fm_v, wid_v, ar_v, azh_v, sw0, sw1, sg0, sg1, so0, so1):
        base0 = _wid() * per_w
        sws, sgs, sos = (sw0, sw1), (sg0, sg1), (so0, so1)

        def wid_copy(c, b):
            return pltpu.make_async_copy(
                fnode_hbm.at[fm_v.at[pl.ds(c * pc, pc)]],
                wid_v.at[b], sws[b])

        def row_copies(c, b):
            return (
                pltpu.make_async_copy(art_hbm.at[wid_v.at[b]],
                                      ar_v.at[b], sgs[b]),
                pltpu.make_async_copy(azht_hbm.at[wid_v.at[b]],
                                      azh_v.at[b], sgs[b]))

        def out_copies(c, b):
            base = base0 + c * pc
            return (
                pltpu.make_async_copy(ar_v.at[b],
                                      ar_hbm.at[pl.ds(base, pc)], sos[b]),
                pltpu.make_async_copy(azh_v.at[b],
                                      azh_hbm.at[pl.ds(base, pc)], sos[b]))

        pltpu.sync_copy(fmess_hbm.at[pl.ds(base0, per_w)], fm_v)
        wid_copy(0, 0).start()
        wid_copy(0, 0).wait()
        for cp in row_copies(0, 0):
            cp.start()

        def outer(c2, _):
            for b in range(2):
                c = c2 * 2 + b
                nb = (b + 1) % 2

                @pl.when(c + 1 < n_chunks)
                def _():
                    wid_copy(c + 1, nb).start()
                    wid_copy(c + 1, nb).wait()
                    @pl.when(c + 1 >= 2)
                    def _():
                        for cp in out_copies(c - 1, nb):
                            cp.wait()
                    for cp in row_copies(c + 1, nb):
                        cp.start()

                for cp in row_copies(c, b):
                    cp.wait()
                for cp in out_copies(c, b):
                    cp.start()
            return 0

        lax.fori_loop(0, n_chunks // 2, outer, 0)
        for cp in out_copies(n_chunks - 2, 0):
            cp.wait()
        for cp in out_copies(n_chunks - 1, 1):
            cp.wait()

    return k(fmess_p, fnode, ar_t, azh_t)


def _sc_sums(mgf, ar, c_tab, ep):
    per_w = ep // NW
    n_chunks = per_w // CHUNK
    nidx = CHUNK * MAX_NB

    @functools.partial(
        pl.kernel,
        out_type=jax.ShapeDtypeStruct((ep, 2 * D), jnp.float32),
        mesh=_MESH,
        scratch_types=[
            pltpu.VMEM((per_w * MAX_NB,), jnp.int32),
            pltpu.VMEM((2, nidx, 2 * D), jnp.float32),
            pltpu.VMEM((2, CHUNK, D), jnp.float32),
            pltpu.VMEM((2, CHUNK, 2 * D), jnp.float32),
            pltpu.SemaphoreType.DMA,
            pltpu.SemaphoreType.DMA,
            pltpu.SemaphoreType.DMA,
            pltpu.SemaphoreType.DMA,
            pltpu.SemaphoreType.DMA,
            pltpu.SemaphoreType.DMA,
        ],
    )
    def k(mgf_hbm, ar_hbm, c_hbm, s_hbm,
          idx_v, rows_v, ar_v, out_v, sg0, sg1, sa0, sa1, so0, so1):
        base0 = _wid() * per_w
        sgs, sas, sos = (sg0, sg1), (sa0, sa1), (so0, so1)

        def out_copy(c, b):
            return pltpu.make_async_copy(
                out_v.at[b], s_hbm.at[pl.ds(base0 + c * CHUNK, CHUNK)],
                sos[b])

        def gather_pair(c, b):
            return (
                pltpu.make_async_copy(
                    c_hbm.at[idx_v.at[pl.ds(c * nidx, nidx)]],
                    rows_v.at[b], sgs[b]),
                pltpu.make_async_copy(
                    ar_hbm.at[pl.ds(base0 + c * CHUNK, CHUNK)],
                    ar_v.at[b], sas[b]))

        def issue(c, b):
            for cp in gather_pair(c, b):
                cp.start()

        pltpu.sync_copy(
            mgf_hbm.at[pl.ds(base0 * MAX_NB, per_w * MAX_NB)], idx_v)
        issue(0, 0)

        def outer(c2, _):
            for b in range(2):
                c = c2 * 2 + b
                nb = (b + 1) % 2

                @pl.when(c + 1 < n_chunks)
                def _():
                    issue(c + 1, nb)

                for cp in gather_pair(c, b):
                    cp.wait()

                @pl.when(c >= 2)
                def _():
                    out_copy(c - 2, b).wait()

                @plsc.parallel_loop(0, CHUNK, unroll=2)
                def _(e):
                    r0 = e * MAX_NB
                    for s in range(D // 16):
                        o = s * 16
                        nar = -ar_v[b, e, pl.ds(o, 16)]
                        acc_s = jnp.zeros((16,), jnp.float32)
                        acc_g = jnp.zeros((16,), jnp.float32)
                        for j in range(MAX_NB):
                            hrow = rows_v[b, r0 + j, pl.ds(o, 16)]
                            hu = rows_v[b, r0 + j, pl.ds(D + o, 16)]
                            u = nar - hu
                            x = jnp.exp(-jnp.abs(u))
                            q = 1.0 + x
                            r = (0.333044 * q - 1.484415) * q + 2.142748
                            r = r * (2.0 - q * r)
                            wneg = x * r
                            w = jnp.where(u > 0.0, wneg, 1.0 - wneg)
                            acc_s = acc_s + hrow
                            acc_g = acc_g + hrow * w
                        out_v[b, e, pl.ds(o, 16)] = acc_s
                        out_v[b, e, pl.ds(D + o, 16)] = acc_g

                out_copy(c, b).start()
            return 0

        lax.fori_loop(0, n_chunks // 2, outer, 0)
        out_copy(n_chunks - 2, 0).wait()
        out_copy(n_chunks - 1, 1).wait()

    return k(mgf, ar, c_tab)


def _tc_dense(s_tab, azh, wz2, wh2, ur, bur, ep, pad_row, blk):
    grid = ep // blk

    def body(s_ref, azh_ref, wz2_ref, wh2_ref, ur_ref, bur_ref, out_ref):
        i = pl.program_id(0)
        sum_h = s_ref[:, :D]
        sum_g = s_ref[:, D:]
        z = jax.nn.sigmoid(azh_ref[:, :D] + jnp.dot(
            sum_h, wz2_ref[...], preferred_element_type=jnp.float32))
        pre = jnp.tanh(azh_ref[:, D:] + jnp.dot(
            sum_g, wh2_ref[...], preferred_element_type=jnp.float32))
        nh = (1.0 - z) * sum_h + z * pre
        rows = i * blk + lax.broadcasted_iota(jnp.int32, (blk, 1), 0)
        is_pad = rows == pad_row
        nh = jnp.where(is_pad, 0.0, nh)
        hu = jnp.where(is_pad, bur_ref[...],
                       jnp.dot(nh, ur_ref[...],
                               preferred_element_type=jnp.float32)
                       + bur_ref[...])
        out_ref[:, :D] = nh
        out_ref[:, D:] = hu

    wspec = pl.BlockSpec((D, D), lambda i: (0, 0))
    return pl.pallas_call(
        body,
        grid=(grid,),
        in_specs=[
            pl.BlockSpec((blk, 2 * D), lambda i: (i, 0)),
            pl.BlockSpec((blk, 2 * D), lambda i: (i, 0)),
            wspec, wspec, wspec,
            pl.BlockSpec((1, D), lambda i: (0, 0)),
        ],
        out_specs=pl.BlockSpec((blk, 2 * D), lambda i: (i, 0)),
        out_shape=jax.ShapeDtypeStruct((ep, 2 * D), jnp.float32),
    )(s_tab, azh, wz2, wh2, ur, bur)


def _sc_node(ngf, fnode_p, hpad, aw, np_):
    per_w = np_ // NW
    pc = 32
    n_chunks = per_w // pc
    nidx = pc * MAX_NB

    @functools.partial(
        pl.kernel,
        out_type=(jax.ShapeDtypeStruct((np_, D), jnp.float32),
                  jax.ShapeDtypeStruct((np_, D), jnp.float32)),
        mesh=_MESH,
        scratch_types=[
            pltpu.VMEM((per_w * MAX_NB,), jnp.int32),
            pltpu.VMEM((per_w,), jnp.int32),
            pltpu.VMEM((2, nidx, D), jnp.float32),
            pltpu.VMEM((2, pc, D), jnp.float32),
            pltpu.VMEM((2, pc, D), jnp.float32),
            pltpu.SemaphoreType.DMA,
            pltpu.SemaphoreType.DMA,
            pltpu.SemaphoreType.DMA,
            pltpu.SemaphoreType.DMA,
        ],
    )
    def k(ngf_hbm, fn_hbm, hpad_hbm, awt_hbm, s2_hbm, awn_hbm,
          idx_v, fn_v, rows_v, aw_v, out_v, sg0, sg1, so0, so1):
        base0 = _wid() * per_w
        sgs, sos = (sg0, sg1), (so0, so1)

        def gathers(c, b):
            cps = [pltpu.make_async_copy(
                awt_hbm.at[fn_v.at[pl.ds(c * pc, pc)]], aw_v.at[b], sgs[b])]
            for h in range(2):
                cps.append(pltpu.make_async_copy(
                    hpad_hbm.at[idx_v.at[pl.ds(c * nidx + h * 128, 128)]],
                    rows_v.at[b, pl.ds(h * 128, 128)], sgs[b]))
            return cps

        def out_copies(c, b):
            base = base0 + c * pc
            return (
                pltpu.make_async_copy(out_v.at[b],
                                      s2_hbm.at[pl.ds(base, pc)], sos[b]),
                pltpu.make_async_copy(aw_v.at[b],
                                      awn_hbm.at[pl.ds(base, pc)], sos[b]))

        pltpu.sync_copy(ngf_hbm.at[pl.ds(base0 * MAX_NB, per_w * MAX_NB)],
                        idx_v)
        pltpu.sync_copy(fn_hbm.at[pl.ds(base0, per_w)], fn_v)
        for cp in gathers(0, 0):
            cp.start()

        def outer(c2, _):
            for b in range(2):
                c = c2 * 2 + b
                nb = (b + 1) % 2

                @pl.when(c + 1 < n_chunks)
                def _():
                    @pl.when(c + 1 >= 2)
                    def _():
                        for cp in out_copies(c - 1, nb):
                            cp.wait()
                    for cp in gathers(c + 1, nb):
                        cp.start()

                for cp in gathers(c, b):
                    cp.wait()

                @plsc.parallel_loop(0, pc, unroll=2)
                def _(n):
                    r0 = n * MAX_NB
                    for s in range(D // 16):
                        o = s * 16
                        acc = jnp.zeros((16,), jnp.float32)
                        for j in range(MAX_NB):
                            acc = acc + rows_v[b, r0 + j, pl.ds(o, 16)]
                        out_v[b, n, pl.ds(o, 16)] = acc

                for cp in out_copies(c, b):
                    cp.start()
            return 0

        lax.fori_loop(0, n_chunks // 2, outer, 0)
        for cp in out_copies(n_chunks - 2, 0):
            cp.wait()
        for cp in out_copies(n_chunks - 1, 1):
            cp.wait()

    return k(ngf, fnode_p, hpad, aw)


def _tc_root(s2, awn, ww2, np_, blk):
    grid = np_ // blk

    def body(s_ref, aw_ref, w_ref, out_ref):
        out_ref[...] = jax.nn.relu(
            aw_ref[...] + jnp.dot(s_ref[...], w_ref[...],
                                  preferred_element_type=jnp.float32))

    return pl.pallas_call(
        body,
        grid=(grid,),
        in_specs=[
            pl.BlockSpec((blk, D), lambda i: (i, 0)),
            pl.BlockSpec((blk, D), lambda i: (i, 0)),
            pl.BlockSpec((D, D), lambda i: (0, 0)),
        ],
        out_specs=pl.BlockSpec((blk, D), lambda i: (i, 0)),
        out_shape=jax.ShapeDtypeStruct((np_, D), jnp.float32),
    )(s2, awn, ww2)


def kernel(fnode, fmess, node_graph, mess_graph, depth, embedding,
           W_z_w, W_z_b, W_r_w, U_r_w, U_r_b, W_h_w, W_h_b, W_w, W_b):
    E = fmess.shape[0]
    N = fnode.shape[0]

    def _pad_to(x, m):
        q = -x % m
        return x + q

    ep = _pad_to(E + 1, 7168)
    np_ = _pad_to(N, NW * 32 * 2)
    pad_row = ep - 1
    blk = 512
    while ep % blk or np_ % blk:
        blk //= 2

    i32 = jnp.int32
    mg = jnp.where(mess_graph == 0, pad_row, mess_graph - 1).astype(i32)
    mgf = jnp.concatenate(
        [mg.reshape(-1), jnp.full(((ep - E) * MAX_NB,), pad_row, i32)])
    ng = jnp.where(node_graph == 0, pad_row, node_graph - 1).astype(i32)
    ngf = jnp.concatenate(
        [ng.reshape(-1), jnp.full(((np_ - N) * MAX_NB,), pad_row, i32)])
    fmess_p = jnp.concatenate([fmess.astype(i32), jnp.zeros((ep - E,), i32)])
    fnode_p = jnp.concatenate([fnode.astype(i32), jnp.zeros((np_ - N,), i32)])

    wz1, wz2 = W_z_w[:D], W_z_w[D:]
    wh1, wh2 = W_h_w[:D], W_h_w[D:]
    ww1, ww2 = W_w[:D], W_w[D:]
    wcat = jnp.concatenate([W_r_w, wz1, wh1, ww1], axis=1)
    bias = jnp.concatenate(
        [jnp.zeros((D,), jnp.float32), W_z_b, W_h_b, W_b]).reshape(1, 4 * D)
    bur = U_r_b.reshape(1, D)

    ar_t, azh_t, aw_t = _prep_tables(embedding, wcat, bias)

    ar, azh = _edge_prep(fmess_p, fnode.astype(i32), ar_t, azh_t, ep)

    c0 = jnp.concatenate(
        [jnp.zeros((ep, D), jnp.float32),
         jnp.broadcast_to(U_r_b, (ep, D))], axis=1)

    def body(_, c_tab):
        s_tab = _sc_sums(mgf, ar, c_tab, ep)
        return _tc_dense(s_tab, azh, wz2, wh2, U_r_w, bur, ep, pad_row, blk)

    c_tab = lax.fori_loop(0, depth, body, c0)

    hpad = c_tab[:, :D]
    s2, awn = _sc_node(ngf, fnode_p, hpad, aw_t, np_)
    root = _tc_root(s2, awn, ww2, np_, blk)

    return c_tab[:E, :D], root[:N]
